# baseline (speedup 1.0000x reference)
import functools

import jax
import jax.numpy as jnp
from jax import lax
from jax.experimental import pallas as pl
from jax.experimental.pallas import tpu as pltpu
from jax.experimental.pallas import tpu_sc as plsc

B = 16384
EMB = 64
RED = 256
MAX_RATING = 5.0

_NC, _NS = 2, 16
_NW = _NC * _NS
_BPW = B // _NW
_CH = 256


@functools.cache
def _make_gather():
    mesh = plsc.VectorSubcoreMesh(core_axis_name="c", subcore_axis_name="s",
                                  num_cores=_NC, num_subcores=_NS)

    @functools.partial(
        pl.kernel,
        mesh=mesh,
        out_type=(
            jax.ShapeDtypeStruct((B, 2 * EMB), jnp.float32),
            jax.ShapeDtypeStruct((B, 2 * EMB), jnp.float32),
        ),
        scratch_types=[
            pltpu.VMEM((_CH,), jnp.int32),
            pltpu.VMEM((_CH,), jnp.int32),
            pltpu.VMEM((_CH, 2 * EMB), jnp.float32),
            pltpu.VMEM((_CH, 2 * EMB), jnp.float32),
            pltpu.SemaphoreType.DMA,
            pltpu.SemaphoreType.DMA,
        ],
    )
    def gather_kernel(uid_hbm, fid_hbm, utab_hbm, ftab_hbm,
                      uout_hbm, fout_hbm,
                      uidx_v, fidx_v, urows_v, frows_v, usem, fsem):
        wid = lax.axis_index("s") * _NC + lax.axis_index("c")
        for c in range(_BPW // _CH):
            base = wid * _BPW + c * _CH
            pltpu.sync_copy(uid_hbm.at[pl.ds(base, _CH)], uidx_v)
            pltpu.sync_copy(fid_hbm.at[pl.ds(base, _CH)], fidx_v)
            cu = pltpu.async_copy(utab_hbm.at[uidx_v], urows_v, usem)
            cf = pltpu.async_copy(ftab_hbm.at[fidx_v], frows_v, fsem)
            cu.wait()
            cf.wait()
            pltpu.sync_copy(urows_v, uout_hbm.at[pl.ds(base, _CH)])
            pltpu.sync_copy(frows_v, fout_hbm.at[pl.ds(base, _CH)])

    return gather_kernel


_BLK = 2048


def _mlp_body(upar_ref, fpar_ref, upair_ref, fpair_ref,
              w1u_ref, w1f_ref, b1_ref, w2_ref, b2_ref,
              w3_ref, b3_ref, o_ref):
    u = jnp.where(upar_ref[...] == 0,
                  upair_ref[:, :EMB], upair_ref[:, EMB:])
    f = jnp.where(fpar_ref[...] == 0,
                  fpair_ref[:, :EMB], fpair_ref[:, EMB:])
    h = jnp.dot(u, w1u_ref[...], preferred_element_type=jnp.float32)
    h += jnp.dot(f, w1f_ref[...], preferred_element_type=jnp.float32)
    h = jnp.maximum(h + b1_ref[...], 0.0)
    h2 = jnp.dot(h, w2_ref[...], preferred_element_type=jnp.float32)
    h2 = jnp.maximum(h2 + b2_ref[...], 0.0)
    z = jnp.sum(h2 * w3_ref[...], axis=1) + b3_ref[0]
    o_ref[...] = MAX_RATING * jax.nn.sigmoid(z)


def _mlp(upar, fpar, upair, fpair, w1u, w1f, b1, w2, b2, w3, b3):
    grid = (B // _BLK,)
    return pl.pallas_call(
        _mlp_body,
        grid=grid,
        in_specs=[
            pl.BlockSpec((_BLK, 1), lambda i: (i, 0)),
            pl.BlockSpec((_BLK, 1), lambda i: (i, 0)),
            pl.BlockSpec((_BLK, 2 * EMB), lambda i: (i, 0)),
            pl.BlockSpec((_BLK, 2 * EMB), lambda i: (i, 0)),
            pl.BlockSpec((EMB, RED), lambda i: (0, 0)),
            pl.BlockSpec((EMB, RED), lambda i: (0, 0)),
            pl.BlockSpec((1, RED), lambda i: (0, 0)),
            pl.BlockSpec((RED, RED // 2), lambda i: (0, 0)),
            pl.BlockSpec((1, RED // 2), lambda i: (0, 0)),
            pl.BlockSpec((1, RED // 2), lambda i: (0, 0)),
            pl.BlockSpec(memory_space=pltpu.SMEM),
        ],
        out_specs=pl.BlockSpec((_BLK,), lambda i: (i,)),
        out_shape=jax.ShapeDtypeStruct((B,), jnp.float32),
    )(upar, fpar, upair, fpair, w1u, w1f, b1, w2, b2, w3, b3)


@jax.jit
def kernel(user_id, film_id, user_table, film_table, W1, b1, W2, b2, W3, b3):
    uid = user_id.astype(jnp.int32)
    fid = film_id.astype(jnp.int32)
    utab2 = jnp.concatenate([user_table[0::2], user_table[1::2]], axis=1)
    ftab2 = jnp.concatenate([film_table[0::2], film_table[1::2]], axis=1)
    upair, fpair = _make_gather()(uid >> 1, fid >> 1, utab2, ftab2)
    upar = (uid & 1).reshape(B, 1)
    fpar = (fid & 1).reshape(B, 1)
    w1t = W1.T
    w1u = w1t[:EMB]
    w1f = w1t[EMB:]
    w2t = W2.T
    w3 = W3.reshape(1, RED // 2)
    return _mlp(upar, fpar, upair, fpair, w1u, w1f, b1.reshape(1, RED),
                w2t, b2.reshape(1, RED // 2), w3, b3)


# speedup vs baseline: 20.1263x; 20.1263x over previous
import functools

import jax
import jax.numpy as jnp
from jax import lax
from jax.experimental import pallas as pl
from jax.experimental.pallas import tpu as pltpu
from jax.experimental.pallas import tpu_sc as plsc

B = 16384
EMB = 64
RED = 256
MAX_RATING = 5.0

_NC, _NS = 2, 16
_NW = _NC * _NS
_CH = B // _NW
_G = 16


@functools.cache
def _make_gather():
    mesh = plsc.VectorSubcoreMesh(core_axis_name="c", subcore_axis_name="s",
                                  num_cores=_NC, num_subcores=_NS)

    @functools.partial(
        pl.kernel,
        mesh=mesh,
        out_type=jax.ShapeDtypeStruct((B * EMB,), jnp.float32),
        scratch_types=[
            pltpu.VMEM((_CH,), jnp.int32),
            pltpu.VMEM((8 * _G, EMB), jnp.float32),
            pltpu.VMEM((8 * _G, EMB), jnp.float32),
            pltpu.SemaphoreType.DMA,
            pltpu.SemaphoreType.DMA,
            pltpu.SemaphoreType.DMA,
        ],
    )
    def gather_kernel(id_hbm, tab_hbm, out_hbm,
                      idx_v, buf0_v, buf1_v, gsem0, gsem1, wsem):
        wid = lax.axis_index("s") * _NC + lax.axis_index("c")
        base = wid * _CH
        pltpu.sync_copy(id_hbm.at[pl.ds(base, _CH)], idx_v)

        def fire(goff, buf, sem):
            v = idx_v[pl.ds(goff, _G)]
            hs = []
            for j in range(_G):
                row = pl.multiple_of((v[j] >> 3) * 8, 8)
                hs.append(pltpu.async_copy(
                    tab_hbm.at[pl.ds(row, 8), :],
                    buf.at[pl.ds(8 * j, 8), :], sem))
            return v, hs

        def drain_write(goff, v, hs, buf):
            for h in hs:
                h.wait()
            ws = []
            for j in range(_G):
                out_off = pl.multiple_of((base + goff + j) * EMB, EMB)
                ws.append(pltpu.async_copy(
                    buf.at[8 * j + (v[j] & 7), :],
                    out_hbm.at[pl.ds(out_off, EMB)], wsem))
            return ws

        def pair(i, carry):
            g0 = pl.multiple_of(2 * i * _G, _G)
            g1 = pl.multiple_of((2 * i + 1) * _G, _G)
            v0, h0 = fire(g0, buf0_v, gsem0)
            v1, h1 = fire(g1, buf1_v, gsem1)
            w0 = drain_write(g0, v0, h0, buf0_v)
            w1 = drain_write(g1, v1, h1, buf1_v)
            for w in w0 + w1:
                w.wait()
            return carry

        lax.fori_loop(0, _CH // (2 * _G), pair, 0)

    return gather_kernel


_BLK = 2048


def _mlp_body(u_ref, f_ref, w1u_ref, w1f_ref, b1_ref, w2_ref, b2_ref,
              w3_ref, b3_ref, o_ref):
    h = jnp.dot(u_ref[...], w1u_ref[...], preferred_element_type=jnp.float32)
    h += jnp.dot(f_ref[...], w1f_ref[...], preferred_element_type=jnp.float32)
    h = jnp.maximum(h + b1_ref[...], 0.0)
    h2 = jnp.dot(h, w2_ref[...], preferred_element_type=jnp.float32)
    h2 = jnp.maximum(h2 + b2_ref[...], 0.0)
    z = jnp.sum(h2 * w3_ref[...], axis=1) + b3_ref[0]
    o_ref[...] = MAX_RATING * jax.nn.sigmoid(z)


def _mlp(u, f, w1u, w1f, b1, w2, b2, w3, b3):
    grid = (B // _BLK,)
    return pl.pallas_call(
        _mlp_body,
        grid=grid,
        in_specs=[
            pl.BlockSpec((_BLK, EMB), lambda i: (i, 0)),
            pl.BlockSpec((_BLK, EMB), lambda i: (i, 0)),
            pl.BlockSpec((EMB, RED), lambda i: (0, 0)),
            pl.BlockSpec((EMB, RED), lambda i: (0, 0)),
            pl.BlockSpec((1, RED), lambda i: (0, 0)),
            pl.BlockSpec((RED, RED // 2), lambda i: (0, 0)),
            pl.BlockSpec((1, RED // 2), lambda i: (0, 0)),
            pl.BlockSpec((1, RED // 2), lambda i: (0, 0)),
            pl.BlockSpec(memory_space=pltpu.SMEM),
        ],
        out_specs=pl.BlockSpec((_BLK,), lambda i: (i,)),
        out_shape=jax.ShapeDtypeStruct((B,), jnp.float32),
    )(u, f, w1u, w1f, b1, w2, b2, w3, b3)


@jax.jit
def kernel(user_id, film_id, user_table, film_table, W1, b1, W2, b2, W3, b3):
    uid = user_id.astype(jnp.int32)
    fid = film_id.astype(jnp.int32)
    g = _make_gather()
    fflat = g(fid, film_table)
    uflat = g(uid, user_table)
    u = uflat.reshape(B, EMB)
    f = fflat.reshape(B, EMB)
    w1t = W1.T
    w1u = w1t[:EMB]
    w1f = w1t[EMB:]
    w2t = W2.T
    w3 = W3.reshape(1, RED // 2)
    return _mlp(u, f, w1u, w1f, b1.reshape(1, RED),
                w2t, b2.reshape(1, RED // 2), w3, b3)
